# baseline (speedup 1.0000x reference)
import functools

import jax
import jax.numpy as jnp
from jax import lax
from jax.experimental import pallas as pl
from jax.experimental.pallas import tpu as pltpu
from jax.experimental.pallas import tpu_sc as plsc

_NC = 2
_NS = 16
_NW = _NC * _NS
_L = 16
_YPAD = 128


def _fwd_body(U, H, T, BLK, data_ref, w_all_ref, linw_ref, linb_ref, y_ref):
    wc = jnp.concatenate(
        [jnp.dot(w_all_ref[u], linw_ref[u * H:(u + 1) * H, :],
                 preferred_element_type=jnp.float32) for u in range(U)],
        axis=0)
    y = jnp.dot(data_ref[...], wc, preferred_element_type=jnp.float32)
    y = y + linb_ref[...][None, :]
    y_ref[...] = jnp.concatenate(
        [y, jnp.zeros((BLK, _YPAD - T), jnp.float32)], axis=1)


def _loss_body(B, T, out_ref, tgt_ref, loss_ref):
    d = out_ref[...] - tgt_ref[...]
    loss_ref[...] = jnp.reshape(jnp.sum(d * d) * (1.0 / (B * T)), (1, 1))


def kernel(data, mem, W, linW, linb, target, sample_id):
    M = mem.shape[1]
    del mem
    U, DIN, H = W.shape
    B = data.shape[0]
    T = linW.shape[1]
    BLK = 4096
    b_per_w = B // _NW
    PART = ((M + _NS - 1) // _NS + 7) // 8 * 8
    MPAD = PART * _NS

    y_pad = pl.pallas_call(
        functools.partial(_fwd_body, U, H, T, BLK),
        grid=(B // BLK,),
        in_specs=[
            pl.BlockSpec((BLK, U * DIN), lambda i: (i, 0)),
            pl.BlockSpec((U, DIN, H), lambda i: (0, 0, 0)),
            pl.BlockSpec((U * H, T), lambda i: (0, 0)),
            pl.BlockSpec((T,), lambda i: (0,)),
        ],
        out_specs=pl.BlockSpec((BLK, _YPAD), lambda i: (i, 0)),
        out_shape=jax.ShapeDtypeStruct((B, _YPAD), jnp.float32),
    )(data, W, linW, linb)

    mesh = plsc.VectorSubcoreMesh(core_axis_name="c", subcore_axis_name="s")

    @functools.partial(
        pl.kernel, mesh=mesh,
        compiler_params=pltpu.CompilerParams(needs_layout_passes=False),
        out_type=jax.ShapeDtypeStruct((B, _YPAD), jnp.float32),
        scratch_types=[
            pltpu.VMEM((B,), jnp.int32),
            pltpu.VMEM((PART,), jnp.int32),
            pltpu.VMEM((b_per_w,), jnp.int32),
            pltpu.VMEM((b_per_w, _YPAD), jnp.float32),
            pltpu.VMEM_SHARED((MPAD,), jnp.int32),
            pltpu.SemaphoreType.DMA,
        ],
    )
    def _sc_route(sid_hbm, y_hbm, out_hbm,
                  sid_v, tab_v, wid_v, rows_v, tab_sh, sem):
        s = lax.axis_index("s")
        c = lax.axis_index("c")
        worker = s * _NC + c
        lo = s * PART
        lanes = lax.iota(jnp.int32, _L)
        nvec = B // _L

        pltpu.sync_copy(sid_hbm, sid_v)

        @plsc.parallel_loop(0, nvec, unroll=8, carry=jnp.int32(0))
        def _ignored(k, z):
            sv = sid_v[pl.ds(k * _L, _L)]
            bv = lanes + k * _L
            idx = sv - lo
            m = (idx >= 0) & (idx < PART)
            plsc.store_scatter(tab_v, [idx], bv, mask=m)
            return z

        def _not_done(err):
            return jnp.sum(err) > 0

        def _sweep(_):
            @plsc.parallel_loop(0, nvec, unroll=8,
                                carry=jnp.zeros((_L,), jnp.int32))
            def err(k, e):
                sv = sid_v[pl.ds(k * _L, _L)]
                bv = lanes + k * _L
                idx = sv - lo
                m = (idx >= 0) & (idx < PART)
                g = plsc.load_gather(tab_v, [idx], mask=m)
                m2 = m & (g < bv)
                plsc.store_scatter(tab_v, [idx], bv, mask=m2)
                return e + plsc.all_reduce_population_count(m2)

            return err

        lax.while_loop(_not_done, _sweep, jnp.ones((_L,), jnp.int32))

        pltpu.sync_copy(tab_v, tab_sh.at[pl.ds(lo, PART)])
        plsc.subcore_barrier()

        base = worker * b_per_w
        pltpu.async_copy(
            tab_sh.at[sid_v.at[pl.ds(base, b_per_w)]], wid_v, sem).wait()
        pltpu.async_copy(y_hbm.at[wid_v], rows_v, sem).wait()
        pltpu.sync_copy(rows_v, out_hbm.at[pl.ds(base, b_per_w)])

    out_pad = _sc_route(sample_id, y_pad)
    out = out_pad[:, :T]

    loss = pl.pallas_call(
        functools.partial(_loss_body, B, T),
        in_specs=[
            pl.BlockSpec((B, T), lambda: (0, 0)),
            pl.BlockSpec((B, T), lambda: (0, 0)),
        ],
        out_specs=pl.BlockSpec((1, 1), lambda: (0, 0)),
        out_shape=jax.ShapeDtypeStruct((1, 1), jnp.float32),
    )(out, target)

    return out, loss[0, 0]


# speedup vs baseline: 1.0252x; 1.0252x over previous
import functools

import jax
import jax.numpy as jnp
from jax import lax
from jax.experimental import pallas as pl
from jax.experimental.pallas import tpu as pltpu
from jax.experimental.pallas import tpu_sc as plsc

_NC = 2
_NS = 16
_NW = _NC * _NS
_L = 16
_YPAD = 128


def _fwd_body(U, H, T, BLK, data_ref, w_all_ref, linw_ref, linb_ref, y_ref):
    wc = jnp.concatenate(
        [jnp.dot(w_all_ref[u], linw_ref[u * H:(u + 1) * H, :],
                 preferred_element_type=jnp.float32) for u in range(U)],
        axis=0)
    y = jnp.dot(data_ref[...], wc, preferred_element_type=jnp.float32)
    y = y + linb_ref[...][None, :]
    y_ref[...] = jnp.concatenate(
        [y, jnp.zeros((BLK, _YPAD - T), jnp.float32)], axis=1)


def _loss_body(B, T, out_ref, tgt_ref, loss_ref):
    d = out_ref[...] - tgt_ref[...]
    loss_ref[...] = jnp.reshape(jnp.sum(d * d) * (1.0 / (B * T)), (1, 1))


def kernel(data, mem, W, linW, linb, target, sample_id):
    M = mem.shape[1]
    del mem
    U, DIN, H = W.shape
    B = data.shape[0]
    T = linW.shape[1]
    BLK = 2048
    b_per_w = B // _NW
    PART = ((M + _NS - 1) // _NS + 7) // 8 * 8
    MPAD = PART * _NS

    y_pad = pl.pallas_call(
        functools.partial(_fwd_body, U, H, T, BLK),
        grid=(B // BLK,),
        in_specs=[
            pl.BlockSpec((BLK, U * DIN), lambda i: (i, 0)),
            pl.BlockSpec((U, DIN, H), lambda i: (0, 0, 0)),
            pl.BlockSpec((U * H, T), lambda i: (0, 0)),
            pl.BlockSpec((T,), lambda i: (0,)),
        ],
        out_specs=pl.BlockSpec((BLK, _YPAD), lambda i: (i, 0)),
        out_shape=jax.ShapeDtypeStruct((B, _YPAD), jnp.float32),
    )(data, W, linW, linb)

    mesh = plsc.VectorSubcoreMesh(core_axis_name="c", subcore_axis_name="s")

    @functools.partial(
        pl.kernel, mesh=mesh,
        compiler_params=pltpu.CompilerParams(needs_layout_passes=False),
        out_type=jax.ShapeDtypeStruct((B, _YPAD), jnp.float32),
        scratch_types=[
            pltpu.VMEM((B,), jnp.int32),
            pltpu.VMEM((PART,), jnp.int32),
            pltpu.VMEM((b_per_w,), jnp.int32),
            pltpu.VMEM((b_per_w, _YPAD), jnp.float32),
            pltpu.VMEM_SHARED((MPAD,), jnp.int32),
            pltpu.SemaphoreType.DMA,
        ],
    )
    def _sc_route(sid_hbm, y_hbm, out_hbm,
                  sid_v, tab_v, wid_v, rows_v, tab_sh, sem):
        s = lax.axis_index("s")
        c = lax.axis_index("c")
        worker = s * _NC + c
        lo = s * PART
        lanes = lax.iota(jnp.int32, _L)
        nvec = B // _L

        pltpu.sync_copy(sid_hbm, sid_v)

        @plsc.parallel_loop(0, nvec, unroll=8, carry=jnp.int32(0))
        def _ignored(k, z):
            sv = sid_v[pl.ds(k * _L, _L)]
            bv = lanes + k * _L
            idx = sv - lo
            m = (idx >= 0) & (idx < PART)
            plsc.store_scatter(tab_v, [idx], bv, mask=m)
            return z

        def _not_done(err):
            return jnp.sum(err) > 0

        def _sweep(_):
            @plsc.parallel_loop(0, nvec, unroll=8,
                                carry=jnp.zeros((_L,), jnp.int32))
            def err(k, e):
                sv = sid_v[pl.ds(k * _L, _L)]
                bv = lanes + k * _L
                idx = sv - lo
                m = (idx >= 0) & (idx < PART)
                g = plsc.load_gather(tab_v, [idx], mask=m)
                m2 = m & (g < bv)
                plsc.store_scatter(tab_v, [idx], bv, mask=m2)
                return e + plsc.all_reduce_population_count(m2)

            return err

        lax.while_loop(_not_done, _sweep, jnp.ones((_L,), jnp.int32))

        pltpu.sync_copy(tab_v, tab_sh.at[pl.ds(lo, PART)])
        plsc.subcore_barrier()

        base = worker * b_per_w
        pltpu.async_copy(
            tab_sh.at[sid_v.at[pl.ds(base, b_per_w)]], wid_v, sem).wait()
        pltpu.async_copy(y_hbm.at[wid_v], rows_v, sem).wait()
        pltpu.sync_copy(rows_v, out_hbm.at[pl.ds(base, b_per_w)])

    out_pad = _sc_route(sample_id, y_pad)
    out = out_pad[:, :T]

    loss = pl.pallas_call(
        functools.partial(_loss_body, B, T),
        in_specs=[
            pl.BlockSpec((B, T), lambda: (0, 0)),
            pl.BlockSpec((B, T), lambda: (0, 0)),
        ],
        out_specs=pl.BlockSpec((1, 1), lambda: (0, 0)),
        out_shape=jax.ShapeDtypeStruct((1, 1), jnp.float32),
    )(out, target)

    return out, loss[0, 0]
